# baseline (speedup 1.0000x reference)
import functools

import jax
import jax.numpy as jnp
from jax.experimental import pallas as pl
from jax.experimental.pallas import tpu as pltpu

_K = 10
_R = 8192


def _topk_body(ids_ref, qrow_ref, blk_ref, pos_ref, neg_ref,
               tv_ref, ti_ref, bv_ref, bi_ref, *, nb, r, n):
    b = pl.program_id(0)

    @pl.when(b == 0)
    def _init():
        tv_ref[...] = jnp.full((_K, 8, 128), -jnp.inf, jnp.float32)
        bv_ref[...] = jnp.full((_K, 8, 128), jnp.inf, jnp.float32)
        ti_ref[...] = jnp.zeros((_K, 8, 128), jnp.int32)
        bi_ref[...] = jnp.zeros((_K, 8, 128), jnp.int32)

    q = qrow_ref[0]
    blk = blk_ref[...]
    s = jax.lax.dot_general(
        q.astype(jnp.bfloat16), blk.astype(jnp.bfloat16),
        (((1,), (1,)), ((), ())),
        preferred_element_type=jnp.float32)
    g = r // 128
    s3 = s.reshape(1, g, 128)
    idx3 = (b * r
            + jax.lax.broadcasted_iota(jnp.int32, (1, g, 128), 1) * 128
            + jax.lax.broadcasted_iota(jnp.int32, (1, g, 128), 2))
    valid = idx3 < n
    s3p = jnp.where(valid, s3, -jnp.inf)
    s3n = jnp.where(valid, s3, jnp.inf)

    tv = [tv_ref[j] for j in range(_K)]
    ti = [ti_ref[j] for j in range(_K)]
    bv = [bv_ref[j] for j in range(_K)]
    bi = [bi_ref[j] for j in range(_K)]

    for t in range(g // 8):
        i0 = idx3[0, t * 8:(t + 1) * 8, :]
        v, i = s3p[0, t * 8:(t + 1) * 8, :], i0
        for j in range(_K):
            m = v > tv[j]
            tv[j], v = jnp.where(m, v, tv[j]), jnp.where(m, tv[j], v)
            ti[j], i = jnp.where(m, i, ti[j]), jnp.where(m, ti[j], i)
        v, i = s3n[0, t * 8:(t + 1) * 8, :], i0
        for j in range(_K):
            m = v < bv[j]
            bv[j], v = jnp.where(m, v, bv[j]), jnp.where(m, bv[j], v)
            bi[j], i = jnp.where(m, i, bi[j]), jnp.where(m, bi[j], i)

    for j in range(_K):
        tv_ref[j] = tv[j]
        ti_ref[j] = ti[j]
        bv_ref[j] = bv[j]
        bi_ref[j] = bi[j]

    @pl.when(b == nb - 1)
    def _final():
        sub = jax.lax.broadcasted_iota(jnp.int32, (8, 128), 0)
        lane = jax.lax.broadcasted_iota(jnp.int32, (8, 128), 1)
        big = jnp.int32(2147483647)

        vals = tv_ref[...]
        idxs = ti_ref[...]
        acc = jnp.zeros((8, 128), jnp.int32)
        for k in range(_K):
            m = jnp.max(vals)
            sel = jnp.min(jnp.where(vals == m, idxs, big))
            vals = jnp.where(idxs == sel, -jnp.inf, vals)
            acc = jnp.where((sub == 0) & (lane == k), sel, acc)
        pos_ref[...] = acc

        vals = bv_ref[...]
        idxs = bi_ref[...]
        acc = jnp.zeros((8, 128), jnp.int32)
        for k in range(_K):
            m = jnp.min(vals)
            sel = jnp.min(jnp.where(vals == m, idxs, big))
            vals = jnp.where(idxs == sel, jnp.inf, vals)
            acc = jnp.where((sub == 0) & (lane == k), sel, acc)
        neg_ref[...] = acc


def _gather_body(ids_ref, emb_ref, out_ref):
    out_ref[...] = emb_ref[...]


def kernel(x, index_vectors, embeddings):
    n, d = index_vectors.shape
    r = min(_R, n)
    nb = -(-n // r)
    anchor_id = x[-1, 0].astype(jnp.int32).reshape(1)

    iv3 = index_vectors.reshape(n, 1, d)
    pos_ids, neg_ids = pl.pallas_call(
        functools.partial(_topk_body, nb=nb, r=r, n=n),
        grid_spec=pltpu.PrefetchScalarGridSpec(
            num_scalar_prefetch=1,
            grid=(nb,),
            in_specs=[
                pl.BlockSpec((1, 1, d), lambda b, ids: (ids[0], 0, 0)),
                pl.BlockSpec((r, d), lambda b, ids: (b, 0)),
            ],
            out_specs=[
                pl.BlockSpec((8, 128), lambda b, ids: (0, 0)),
                pl.BlockSpec((8, 128), lambda b, ids: (0, 0)),
            ],
            scratch_shapes=[
                pltpu.VMEM((_K, 8, 128), jnp.float32),
                pltpu.VMEM((_K, 8, 128), jnp.int32),
                pltpu.VMEM((_K, 8, 128), jnp.float32),
                pltpu.VMEM((_K, 8, 128), jnp.int32),
            ],
        ),
        out_shape=[
            jax.ShapeDtypeStruct((8, 128), jnp.int32),
            jax.ShapeDtypeStruct((8, 128), jnp.int32),
        ],
    )(anchor_id, iv3, index_vectors)

    gather_ids = jnp.concatenate(
        [anchor_id, pos_ids[0, :_K], neg_ids[0, :_K]])

    emb3 = embeddings.reshape(n, 1, d)
    rows = pl.pallas_call(
        _gather_body,
        grid_spec=pltpu.PrefetchScalarGridSpec(
            num_scalar_prefetch=1,
            grid=(2 * _K + 1,),
            in_specs=[pl.BlockSpec((1, 1, d), lambda i, ids: (ids[i], 0, 0))],
            out_specs=pl.BlockSpec((1, 1, d), lambda i, ids: (i, 0, 0)),
        ),
        out_shape=jax.ShapeDtypeStruct((2 * _K + 1, 1, d), jnp.float32),
    )(gather_ids, emb3)

    anchor = rows[0, 0, :]
    positive = rows[1:_K + 1, 0, :][None]
    negative = rows[_K + 1:, 0, :][None]
    return (anchor, positive, negative)
